# baseline (speedup 1.0000x reference)
import functools

import jax
import jax.numpy as jnp
from jax.experimental import pallas as pl
from jax.experimental.pallas import tpu as pltpu


def _flash_body(q_ref, k_ref, v_ref, h_ref, o_ref,
                qn_ref, m_ref, d_ref, acc_ref):
    i = pl.program_id(0)
    nsteps = pl.num_programs(0)

    @pl.when(i == 0)
    def _init():
        q = q_ref[...]
        qn = q / jnp.maximum(
            jnp.sqrt(jnp.sum(q * q, axis=1, keepdims=True)), 1e-12)
        qn_ref[...] = qn
        m_ref[...] = jnp.full_like(m_ref, -jnp.inf)
        d_ref[...] = jnp.zeros_like(d_ref)
        acc_ref[...] = jnp.zeros_like(acc_ref)

    k = k_ref[...]
    qn = qn_ref[...]
    raw = jax.lax.dot_general(
        qn, k, (((1,), (1,)), ((), ())),
        preferred_element_type=jnp.float32,
        precision=jax.lax.Precision.HIGHEST)
    ones = jnp.ones((1, k.shape[1]), jnp.float32)
    sumsq = jax.lax.dot_general(
        ones, k * k, (((1,), (1,)), ((), ())),
        preferred_element_type=jnp.float32,
        precision=jax.lax.Precision.HIGHEST)
    inv_norm = 1.0 / jnp.maximum(jnp.sqrt(sumsq), 1e-12)
    scale = h_ref[0] * inv_norm * 10.0
    scores = raw * scale

    m_prev = m_ref[...]
    m_new = jnp.maximum(m_prev, jnp.max(scores, axis=1, keepdims=True))
    alpha = jnp.exp(m_prev - m_new)
    p = jnp.exp(scores - m_new)
    m_ref[...] = m_new
    d_ref[...] = d_ref[...] * alpha + jnp.sum(p, axis=1, keepdims=True)
    pv = jax.lax.dot_general(
        p, v_ref[...], (((1,), (0,)), ((), ())),
        preferred_element_type=jnp.float32,
        precision=jax.lax.Precision.HIGHEST)
    acc_ref[...] = acc_ref[...] * alpha + pv

    @pl.when(i == nsteps - 1)
    def _done():
        o_ref[...] = acc_ref[...] / d_ref[...]


@functools.partial(jax.jit, static_argnames=("interpret",))
def kernel(latent, keys, vals, hardness, interpret=False):
    b, l, d = latent.shape
    s = keys.shape[0]
    nq = b * l
    q = latent.reshape(nq, d)
    chunk = 5000 if s % 5000 == 0 else s
    grid = (s // chunk,)
    h3 = hardness.reshape(s // chunk, 1, chunk)
    out = pl.pallas_call(
        _flash_body,
        grid=grid,
        in_specs=[
            pl.BlockSpec((nq, d), lambda i: (0, 0)),
            pl.BlockSpec((chunk, d), lambda i: (i, 0)),
            pl.BlockSpec((chunk, d), lambda i: (i, 0)),
            pl.BlockSpec((1, 1, chunk), lambda i: (i, 0, 0)),
        ],
        out_specs=pl.BlockSpec((nq, d), lambda i: (0, 0)),
        out_shape=jax.ShapeDtypeStruct((nq, d), jnp.float32),
        scratch_shapes=[
            pltpu.VMEM((nq, d), jnp.float32),
            pltpu.VMEM((nq, 1), jnp.float32),
            pltpu.VMEM((nq, 1), jnp.float32),
            pltpu.VMEM((nq, d), jnp.float32),
        ],
        interpret=interpret,
    )(q, keys, vals, h3)
    return out.reshape(b, l, d)


# speedup vs baseline: 1.7152x; 1.7152x over previous
import functools

import jax
import jax.numpy as jnp
from jax.experimental import pallas as pl
from jax.experimental.pallas import tpu as pltpu


def _flash_body(q_ref, k_ref, v_ref, h_ref, o_ref,
                qn_ref, m_ref, d_ref, acc_ref):
    i = pl.program_id(0)
    nsteps = pl.num_programs(0)

    @pl.when(i == 0)
    def _init():
        q = q_ref[...]
        qn = q / jnp.maximum(
            jnp.sqrt(jnp.sum(q * q, axis=1, keepdims=True)), 1e-12)
        qn_ref[...] = qn
        m_ref[...] = jnp.full_like(m_ref, -jnp.inf)
        d_ref[...] = jnp.zeros_like(d_ref)
        acc_ref[...] = jnp.zeros_like(acc_ref)

    k = k_ref[...]
    qn = qn_ref[...]
    raw = jax.lax.dot_general(
        qn, k, (((1,), (1,)), ((), ())),
        preferred_element_type=jnp.float32)
    ones = jnp.ones((1, k.shape[1]), jnp.float32)
    sumsq = jax.lax.dot_general(
        ones, k * k, (((1,), (1,)), ((), ())),
        preferred_element_type=jnp.float32)
    inv_norm = 1.0 / jnp.maximum(jnp.sqrt(sumsq), 1e-12)
    scale = h_ref[0] * inv_norm * 10.0
    scores = raw * scale

    m_prev = m_ref[...]
    m_new = jnp.maximum(m_prev, jnp.max(scores, axis=1, keepdims=True))
    alpha = jnp.exp(m_prev - m_new)
    p = jnp.exp(scores - m_new)
    m_ref[...] = m_new
    d_ref[...] = d_ref[...] * alpha + jnp.sum(p, axis=1, keepdims=True)
    pv = jax.lax.dot_general(
        p, v_ref[...], (((1,), (0,)), ((), ())),
        preferred_element_type=jnp.float32)
    acc_ref[...] = acc_ref[...] * alpha + pv

    @pl.when(i == nsteps - 1)
    def _done():
        o_ref[...] = acc_ref[...] / d_ref[...]


@functools.partial(jax.jit, static_argnames=("interpret",))
def kernel(latent, keys, vals, hardness, interpret=False):
    b, l, d = latent.shape
    s = keys.shape[0]
    nq = b * l
    q = latent.reshape(nq, d)
    chunk = 5000 if s % 5000 == 0 else s
    grid = (s // chunk,)
    h3 = hardness.reshape(s // chunk, 1, chunk)
    out = pl.pallas_call(
        _flash_body,
        grid=grid,
        in_specs=[
            pl.BlockSpec((nq, d), lambda i: (0, 0)),
            pl.BlockSpec((chunk, d), lambda i: (i, 0)),
            pl.BlockSpec((chunk, d), lambda i: (i, 0)),
            pl.BlockSpec((1, 1, chunk), lambda i: (i, 0, 0)),
        ],
        out_specs=pl.BlockSpec((nq, d), lambda i: (0, 0)),
        out_shape=jax.ShapeDtypeStruct((nq, d), jnp.float32),
        scratch_shapes=[
            pltpu.VMEM((nq, d), jnp.float32),
            pltpu.VMEM((nq, 1), jnp.float32),
            pltpu.VMEM((nq, 1), jnp.float32),
            pltpu.VMEM((nq, d), jnp.float32),
        ],
        interpret=interpret,
    )(q, keys, vals, h3)
    return out.reshape(b, l, d)


# speedup vs baseline: 1.7759x; 1.0354x over previous
import functools

import jax
import jax.numpy as jnp
from jax.experimental import pallas as pl
from jax.experimental.pallas import tpu as pltpu


def _flash_body(q_ref, k_ref, v_ref, h_ref, o_ref,
                qn_ref, m_ref, d_ref, acc_ref):
    i = pl.program_id(0)
    nsteps = pl.num_programs(0)

    @pl.when(i == 0)
    def _init():
        q = q_ref[...]
        qn = q / jnp.maximum(
            jnp.sqrt(jnp.sum(q * q, axis=1, keepdims=True)), 1e-12)
        qn_ref[...] = qn
        m_ref[...] = jnp.full_like(m_ref, -jnp.inf)
        d_ref[...] = jnp.zeros_like(d_ref)
        acc_ref[...] = jnp.zeros_like(acc_ref)

    k = k_ref[...]
    qn = qn_ref[...]
    raw = jax.lax.dot_general(
        qn, k, (((1,), (1,)), ((), ())),
        preferred_element_type=jnp.float32)
    ones = jnp.ones((1, k.shape[1]), jnp.float32)
    sumsq = jax.lax.dot_general(
        ones, k * k, (((1,), (1,)), ((), ())),
        preferred_element_type=jnp.float32)
    inv_norm = 1.0 / jnp.maximum(jnp.sqrt(sumsq), 1e-12)
    scale = h_ref[0] * inv_norm * 10.0
    scores = raw * scale

    m_prev = m_ref[...]
    m_new = jnp.maximum(m_prev, jnp.max(scores, axis=1, keepdims=True))
    alpha = jnp.exp(m_prev - m_new)
    p = jnp.exp(scores - m_new)
    m_ref[...] = m_new
    d_ref[...] = d_ref[...] * alpha + jnp.sum(p, axis=1, keepdims=True)
    pv = jax.lax.dot_general(
        p, v_ref[...], (((1,), (0,)), ((), ())),
        preferred_element_type=jnp.float32)
    acc_ref[...] = acc_ref[...] * alpha + pv

    @pl.when(i == nsteps - 1)
    def _done():
        o_ref[...] = acc_ref[...] / d_ref[...]


@functools.partial(jax.jit, static_argnames=("interpret",))
def kernel(latent, keys, vals, hardness, interpret=False):
    b, l, d = latent.shape
    s = keys.shape[0]
    nq = b * l
    q = latent.reshape(nq, d)
    chunk = 10000 if s % 10000 == 0 else s
    grid = (s // chunk,)
    h3 = hardness.reshape(s // chunk, 1, chunk)
    out = pl.pallas_call(
        _flash_body,
        grid=grid,
        in_specs=[
            pl.BlockSpec((nq, d), lambda i: (0, 0)),
            pl.BlockSpec((chunk, d), lambda i: (i, 0)),
            pl.BlockSpec((chunk, d), lambda i: (i, 0)),
            pl.BlockSpec((1, 1, chunk), lambda i: (i, 0, 0)),
        ],
        out_specs=pl.BlockSpec((nq, d), lambda i: (0, 0)),
        out_shape=jax.ShapeDtypeStruct((nq, d), jnp.float32),
        scratch_shapes=[
            pltpu.VMEM((nq, d), jnp.float32),
            pltpu.VMEM((nq, 1), jnp.float32),
            pltpu.VMEM((nq, 1), jnp.float32),
            pltpu.VMEM((nq, d), jnp.float32),
        ],
        interpret=interpret,
    )(q, keys, vals, h3)
    return out.reshape(b, l, d)
